# baseline (speedup 1.0000x reference)
import jax
import jax.numpy as jnp
from jax.experimental import pallas as pl

SEQ_LEN = 2048
BT_SIZE = 4
D_MODEL = 4096


def _concat_kernel(x_ref, p_ref, o_ref):
    o_ref[:, :, :D_MODEL] = x_ref[...]
    o_ref[:, :, D_MODEL:] = p_ref[...][None, :, :]


def kernel(inputs, pos_table):
    R = 256
    grid = (BT_SIZE, SEQ_LEN // R)
    return pl.pallas_call(
        _concat_kernel,
        grid=grid,
        in_specs=[
            pl.BlockSpec((1, R, D_MODEL), lambda b, s: (b, s, 0)),
            pl.BlockSpec((R, 1), lambda b, s: (s, 0)),
        ],
        out_specs=pl.BlockSpec((1, R, D_MODEL + 1), lambda b, s: (b, s, 0)),
        out_shape=jax.ShapeDtypeStruct((BT_SIZE, SEQ_LEN, D_MODEL + 1), jnp.float32),
    )(inputs, pos_table)
